# baseline (speedup 1.0000x reference)
import functools

import jax
import jax.numpy as jnp
from jax import lax
from jax.experimental import pallas as pl
from jax.experimental.pallas import tpu as pltpu
from jax.experimental.pallas import tpu_sc as plsc

VOCAB = 100000
D = 128
S = 200
B = 4096
FLAT = B * S
NC, NS, L = 2, 16, 16
NW = NC * NS
PER_W = FLAT // NW
CHUNK = S
NCHUNK = PER_W // CHUNK
NBUF = 4
SCALE = float(D) ** 0.5


def _positional_encoding(seq_len, d_model):
    position = jnp.arange(0, seq_len, dtype=jnp.float32)[:, None]
    _2i = jnp.arange(0, d_model, 2, dtype=jnp.float32)
    angle = position / jnp.power(10000.0, _2i / d_model)
    enc = jnp.zeros((seq_len, d_model), dtype=jnp.float32)
    enc = enc.at[:, 0::2].set(jnp.sin(angle))
    enc = enc.at[:, 1::2].set(jnp.cos(angle))
    return enc


def _make_sc_kernel():
    mesh = plsc.VectorSubcoreMesh(core_axis_name="c", subcore_axis_name="s",
                                  num_cores=NC, num_subcores=NS)

    @functools.partial(
        pl.kernel,
        out_type=jax.ShapeDtypeStruct((FLAT, D), jnp.float32),
        mesh=mesh,
        scratch_types=[
            pltpu.VMEM((S, D), jnp.float32),
            tuple(pltpu.VMEM((CHUNK, D), jnp.float32) for _ in range(NBUF)),
            tuple(pltpu.VMEM((CHUNK,), jnp.int32) for _ in range(NBUF)),
            tuple(pltpu.SemaphoreType.DMA for _ in range(NBUF)),
            tuple(pltpu.SemaphoreType.DMA for _ in range(NBUF)),
            tuple(pltpu.SemaphoreType.DMA for _ in range(NBUF)),
        ],
    )
    def emb_kernel(table_hbm, idx_hbm, pe_hbm, out_hbm,
                   pe_v, bufs, idxs, gsems, psems, isems):
        wid = lax.axis_index("s") * NC + lax.axis_index("c")
        base = wid * PER_W

        pltpu.sync_copy(pe_hbm, pe_v)

        def start_idx(g, s):
            pltpu.async_copy(
                idx_hbm.at[pl.ds(base + g * CHUNK, CHUNK)], idxs[s], isems[s])

        def wait_idx(s):
            pltpu.make_async_copy(
                idx_hbm.at[pl.ds(base, CHUNK)], idxs[s], isems[s]).wait()

        def start_gather(g, s):
            pltpu.async_copy(
                table_hbm.at[pl.ds(0, CHUNK)], bufs[s], gsems[s])

        def wait_gather(s):
            pltpu.make_async_copy(
                table_hbm.at[pl.ds(0, CHUNK)], bufs[s], gsems[s]).wait()

        def start_put(g, s):
            pltpu.async_copy(
                bufs[s], out_hbm.at[pl.ds(base + g * CHUNK, CHUNK)], psems[s])

        def wait_put(s):
            pltpu.make_async_copy(
                bufs[s], out_hbm.at[pl.ds(base, CHUNK)], psems[s]).wait()

        def compute(s):
            buf = bufs[s]

            @pl.loop(0, CHUNK)
            def _(r):
                for j in range(D // L):
                    sl = pl.ds(j * L, L)
                    buf[r, sl] = buf[r, sl] * SCALE + pe_v[r, sl]

        def step(g, s, idx_g=None, gather_g=None, put_wait=True):
            wait_gather(s)
            compute(s)
            start_put(g, s)
            if idx_g is not None:
                start_idx(idx_g, s)
            if gather_g is not None:
                s2 = (s + 2) % NBUF
                if put_wait:
                    wait_put(s2)
                wait_idx(s2)
                start_gather(gather_g, s2)

        for b in range(NBUF):
            start_idx(b, b)
        for b in range(2):
            wait_idx(b)
            start_gather(b, b)
        step(0, 0, idx_g=4, gather_g=2, put_wait=False)
        step(1, 1, idx_g=5, gather_g=3, put_wait=False)
        step(2, 2, idx_g=6, gather_g=4, put_wait=True)
        step(3, 3, idx_g=7, gather_g=5, put_wait=True)

        @pl.loop(1, NCHUNK // NBUF - 1)
        def _(i):
            g0 = i * NBUF
            for b in range(NBUF):
                step(g0 + b, b, idx_g=g0 + b + NBUF, gather_g=g0 + b + 2)

        step(NCHUNK - 4, 0, gather_g=NCHUNK - 2)
        step(NCHUNK - 3, 1, gather_g=NCHUNK - 1)
        step(NCHUNK - 2, 2)
        step(NCHUNK - 1, 3)
        for b in range(NBUF):
            wait_put(b)

    return emb_kernel


_make_sc_kernel = functools.cache(_make_sc_kernel)


@jax.jit
def kernel(x, table):
    idx = x.reshape(-1).astype(jnp.int32)
    pe = _positional_encoding(S, D)
    out = _make_sc_kernel()(table, idx, pe)
    return out.reshape(B, S, D)


# speedup vs baseline: 3.0782x; 3.0782x over previous
import functools

import jax
import jax.numpy as jnp
from jax import lax
from jax.experimental import pallas as pl
from jax.experimental.pallas import tpu as pltpu
from jax.experimental.pallas import tpu_sc as plsc

VOCAB = 100000
D = 128
S = 200
B = 4096
FLAT = B * S
NC, NS, L = 2, 16, 16
NW = NC * NS
PER_W = FLAT // NW
CHUNK = S
NCHUNK = PER_W // CHUNK
NBUF = 4
SCALE = float(D) ** 0.5


def _positional_encoding(seq_len, d_model):
    position = jnp.arange(0, seq_len, dtype=jnp.float32)[:, None]
    _2i = jnp.arange(0, d_model, 2, dtype=jnp.float32)
    angle = position / jnp.power(10000.0, _2i / d_model)
    enc = jnp.zeros((seq_len, d_model), dtype=jnp.float32)
    enc = enc.at[:, 0::2].set(jnp.sin(angle))
    enc = enc.at[:, 1::2].set(jnp.cos(angle))
    return enc


def _make_sc_kernel():
    mesh = plsc.VectorSubcoreMesh(core_axis_name="c", subcore_axis_name="s",
                                  num_cores=NC, num_subcores=NS)

    @functools.partial(
        pl.kernel,
        out_type=jax.ShapeDtypeStruct((FLAT, D), jnp.float32),
        mesh=mesh,
        scratch_types=[
            pltpu.VMEM((S, D), jnp.float32),
            tuple(pltpu.VMEM((CHUNK, D), jnp.float32) for _ in range(NBUF)),
            tuple(pltpu.VMEM((CHUNK,), jnp.int32) for _ in range(NBUF)),
            tuple(pltpu.SemaphoreType.DMA for _ in range(NBUF)),
            tuple(pltpu.SemaphoreType.DMA for _ in range(NBUF)),
            tuple(pltpu.SemaphoreType.DMA for _ in range(NBUF)),
        ],
    )
    def emb_kernel(table_hbm, idx_hbm, pe_hbm, out_hbm,
                   pe_v, bufs, idxs, gsems, psems, isems):
        wid = lax.axis_index("s") * NC + lax.axis_index("c")
        base = wid * PER_W

        pltpu.sync_copy(pe_hbm, pe_v)

        def start_idx(g, s):
            pltpu.async_copy(
                idx_hbm.at[pl.ds(base + g * CHUNK, CHUNK)], idxs[s], isems[s])

        def wait_idx(s):
            pltpu.make_async_copy(
                idx_hbm.at[pl.ds(base, CHUNK)], idxs[s], isems[s]).wait()

        def start_gather(g, s):
            pltpu.async_copy(table_hbm.at[idxs[s]], bufs[s], gsems[s])

        def wait_gather(s):
            pltpu.make_async_copy(
                table_hbm.at[idxs[s]], bufs[s], gsems[s]).wait()

        def start_put(g, s):
            pltpu.async_copy(
                bufs[s], out_hbm.at[pl.ds(base + g * CHUNK, CHUNK)], psems[s])

        def wait_put(s):
            pltpu.make_async_copy(
                bufs[s], out_hbm.at[pl.ds(base, CHUNK)], psems[s]).wait()

        def compute(s):
            buf = bufs[s]

            @pl.loop(0, CHUNK)
            def _(r):
                for j in range(D // L):
                    sl = pl.ds(j * L, L)
                    buf[r, sl] = buf[r, sl] * SCALE + pe_v[r, sl]

        def step(g, s, idx_g=None, gather_g=None, put_wait=True):
            wait_gather(s)
            compute(s)
            start_put(g, s)
            if idx_g is not None:
                start_idx(idx_g, s)
            if gather_g is not None:
                s2 = (s + 3) % NBUF
                if put_wait:
                    wait_put(s2)
                wait_idx(s2)
                start_gather(gather_g, s2)

        for b in range(NBUF):
            start_idx(b, b)
        for b in range(3):
            wait_idx(b)
            start_gather(b, b)
        step(0, 0, idx_g=4, gather_g=3, put_wait=False)
        step(1, 1, idx_g=5, gather_g=4, put_wait=True)
        step(2, 2, idx_g=6, gather_g=5, put_wait=True)
        step(3, 3, idx_g=7, gather_g=6, put_wait=True)

        @pl.loop(1, NCHUNK // NBUF - 1)
        def _(i):
            g0 = i * NBUF
            for b in range(NBUF):
                step(g0 + b, b, idx_g=g0 + b + NBUF, gather_g=g0 + b + 3)

        step(NCHUNK - 4, 0, gather_g=NCHUNK - 1)
        step(NCHUNK - 3, 1)
        step(NCHUNK - 2, 2)
        step(NCHUNK - 1, 3)
        for b in range(NBUF):
            wait_put(b)

    return emb_kernel


_make_sc_kernel = functools.cache(_make_sc_kernel)


@jax.jit
def kernel(x, table):
    idx = x.reshape(-1).astype(jnp.int32)
    pe = _positional_encoding(S, D)
    out = _make_sc_kernel()(table, idx, pe)
    return out.reshape(B, S, D)


# speedup vs baseline: 3.1241x; 1.0149x over previous
import functools

import jax
import jax.numpy as jnp
from jax import lax
from jax.experimental import pallas as pl
from jax.experimental.pallas import tpu as pltpu
from jax.experimental.pallas import tpu_sc as plsc

VOCAB = 100000
D = 128
S = 200
B = 4096
FLAT = B * S
NC, NS, L = 2, 16, 16
NW = NC * NS
PER_W = FLAT // NW
CHUNK = S
NCHUNK = PER_W // CHUNK
NBUF = 4
SCALE = float(D) ** 0.5


def _positional_encoding(seq_len, d_model):
    position = jnp.arange(0, seq_len, dtype=jnp.float32)[:, None]
    _2i = jnp.arange(0, d_model, 2, dtype=jnp.float32)
    angle = position / jnp.power(10000.0, _2i / d_model)
    enc = jnp.zeros((seq_len, d_model), dtype=jnp.float32)
    enc = enc.at[:, 0::2].set(jnp.sin(angle))
    enc = enc.at[:, 1::2].set(jnp.cos(angle))
    return enc


def _make_sc_kernel():
    mesh = plsc.VectorSubcoreMesh(core_axis_name="c", subcore_axis_name="s",
                                  num_cores=NC, num_subcores=NS)

    @functools.partial(
        pl.kernel,
        out_type=jax.ShapeDtypeStruct((FLAT, D), jnp.float32),
        mesh=mesh,
        scratch_types=[
            pltpu.VMEM((S, D), jnp.float32),
            tuple(pltpu.VMEM((CHUNK, D), jnp.float32) for _ in range(NBUF)),
            tuple(pltpu.VMEM((CHUNK,), jnp.int32) for _ in range(NBUF)),
            tuple(pltpu.SemaphoreType.DMA for _ in range(NBUF)),
            tuple(pltpu.SemaphoreType.DMA for _ in range(NBUF)),
            tuple(pltpu.SemaphoreType.DMA for _ in range(NBUF)),
        ],
    )
    def emb_kernel(table_hbm, idx_hbm, pe_hbm, out_hbm,
                   pe_v, bufs, idxs, gsems, psems, isems):
        wid = lax.axis_index("s") * NC + lax.axis_index("c")
        base = wid * PER_W

        pltpu.sync_copy(pe_hbm, pe_v)

        def start_idx(g, s):
            pltpu.async_copy(
                idx_hbm.at[pl.ds(base + g * CHUNK, CHUNK)], idxs[s], isems[s])

        def wait_idx(s):
            pltpu.make_async_copy(
                idx_hbm.at[pl.ds(base, CHUNK)], idxs[s], isems[s]).wait()

        def start_gather(g, s):
            pltpu.async_copy(table_hbm.at[idxs[s]], bufs[s], gsems[s])

        def wait_gather(s):
            pltpu.make_async_copy(
                table_hbm.at[idxs[s]], bufs[s], gsems[s]).wait()

        def start_put(g, s):
            pltpu.async_copy(
                bufs[s], out_hbm.at[pl.ds(base + g * CHUNK, CHUNK)], psems[s])

        def wait_put(s):
            pltpu.make_async_copy(
                bufs[s], out_hbm.at[pl.ds(base, CHUNK)], psems[s]).wait()

        def compute(s):
            buf = bufs[s]

            @pl.loop(0, CHUNK)
            def _(r):
                for j in range(D // L):
                    sl = pl.ds(j * L, L)
                    buf[r, sl] = buf[r, sl] * SCALE + pe_v[r, sl]

        def step(g, s, idx_g=None, gather_g=None, put_wait=True):
            wait_gather(s)
            compute(s)
            start_put(g, s)
            if idx_g is not None:
                start_idx(idx_g, s)
            if gather_g is not None:
                s2 = (s + 2) % NBUF
                if put_wait:
                    wait_put(s2)
                wait_idx(s2)
                start_gather(gather_g, s2)

        for b in range(NBUF):
            start_idx(b, b)
        for b in range(2):
            wait_idx(b)
            start_gather(b, b)
        step(0, 0, idx_g=4, gather_g=2, put_wait=False)
        step(1, 1, idx_g=5, gather_g=3, put_wait=False)
        step(2, 2, idx_g=6, gather_g=4, put_wait=True)
        step(3, 3, idx_g=7, gather_g=5, put_wait=True)

        @pl.loop(1, NCHUNK // NBUF - 1)
        def _(i):
            g0 = i * NBUF
            for b in range(NBUF):
                step(g0 + b, b, idx_g=g0 + b + NBUF, gather_g=g0 + b + 2)

        step(NCHUNK - 4, 0, gather_g=NCHUNK - 2)
        step(NCHUNK - 3, 1, gather_g=NCHUNK - 1)
        step(NCHUNK - 2, 2)
        step(NCHUNK - 1, 3)
        for b in range(NBUF):
            wait_put(b)

    return emb_kernel


_make_sc_kernel = functools.cache(_make_sc_kernel)


@jax.jit
def kernel(x, table):
    idx = x.reshape(-1).astype(jnp.int32)
    pe = _positional_encoding(S, D)
    out = _make_sc_kernel()(table, idx, pe)
    return out.reshape(B, S, D)
